# baseline (speedup 1.0000x reference)
import functools

import jax
import jax.numpy as jnp
from jax.experimental import pallas as pl
from jax.experimental.pallas import tpu as pltpu

_LANES = 128
_SUB = 8
_NUM_CORES = 1
_TARGET_TILE_ELEMS = 1024 * 1024


def _nmse_tile_kernel(p_ref, t_ref, o_ref, acc_ref, *, steps_per_core, tile_r,
                      acc_cols):
    i = pl.program_id(1)

    cols = p_ref.shape[1]
    ssd = st = st2 = None
    first = True
    for k in range(tile_r // _SUB):
        band = slice(k * _SUB, (k + 1) * _SUB)
        for c in range(cols // acc_cols):
            cs = slice(c * acc_cols, (c + 1) * acc_cols)
            ps = p_ref[band, cs].astype(jnp.float32)
            ts = t_ref[band, cs].astype(jnp.float32)
            ds = ps - ts
            if first:
                ssd, st, st2 = ds * ds, ts, ts * ts
                first = False
            else:
                ssd += ds * ds
                st += ts
                st2 += ts * ts

    @pl.when(i == 0)
    def _start():
        acc_ref[0] = ssd
        acc_ref[1] = st
        acc_ref[2] = st2

    @pl.when(i > 0)
    def _accum():
        acc_ref[0] += ssd
        acc_ref[1] += st
        acc_ref[2] += st2

    @pl.when(i == steps_per_core - 1)
    def _emit():
        o_ref[0, 0] = jnp.full((_SUB, _LANES), jnp.sum(acc_ref[0]), jnp.float32)
        o_ref[0, 1] = jnp.full((_SUB, _LANES), jnp.sum(acc_ref[1]), jnp.float32)
        o_ref[0, 2] = jnp.full((_SUB, _LANES), jnp.sum(acc_ref[2]), jnp.float32)


def _nmse_partials(p2d, t2d, steps_per_core, tile_r, acc_cols):
    cols = p2d.shape[1]
    kernel_fn = functools.partial(
        _nmse_tile_kernel, steps_per_core=steps_per_core, tile_r=tile_r,
        acc_cols=acc_cols)

    def in_map(c, i):
        return (c * steps_per_core + i, 0)

    n = p2d.size
    bytes_in = n * (p2d.dtype.itemsize + t2d.dtype.itemsize)
    return pl.pallas_call(
        kernel_fn,
        out_shape=jax.ShapeDtypeStruct((_NUM_CORES, 3, _SUB, _LANES), jnp.float32),
        grid=(_NUM_CORES, steps_per_core),
        in_specs=[
            pl.BlockSpec((tile_r, cols), in_map),
            pl.BlockSpec((tile_r, cols), in_map),
        ],
        out_specs=pl.BlockSpec((1, 3, _SUB, _LANES), lambda c, i: (c, 0, 0, 0)),
        scratch_shapes=[pltpu.VMEM((3, _SUB, acc_cols), jnp.float32)],
        compiler_params=pltpu.CompilerParams(
            dimension_semantics=("parallel", "arbitrary"),
        ),
        cost_estimate=pl.CostEstimate(
            flops=6 * n, transcendentals=0, bytes_accessed=bytes_in),
    )(p2d, t2d)


def _pick_tile_r(rows):
    per_core = rows // _NUM_CORES
    best = _SUB
    tr = _SUB
    while tr <= per_core:
        if per_core % tr == 0:
            best = tr
        tr *= 2
    return best


def kernel(prediction: jax.Array, target: jax.Array) -> jax.Array:
    assert prediction.shape == target.shape
    n = prediction.size
    assert n > 0

    p = prediction
    t = target

    if not (p.ndim == 2 and p.shape[0] % (_NUM_CORES * _SUB) == 0
            and p.shape[1] % _LANES == 0):
        p = p.reshape(-1)
        t = t.reshape(-1)
        chunk = _NUM_CORES * _SUB * _LANES
        rem = p.size % chunk
        if rem:
            p = jnp.pad(p, (0, chunk - rem))
            t = jnp.pad(t, (0, chunk - rem))
        p = p.reshape(-1, _LANES)
        t = t.reshape(-1, _LANES)

    rows, cols = p.shape
    tile_r = _pick_tile_r(rows)
    while tile_r > _SUB and tile_r * cols > _TARGET_TILE_ELEMS:
        tile_r //= 2
    steps_per_core = rows // (_NUM_CORES * tile_r)

    acc_cols = cols
    for cand in (1024, 512, 256, 128):
        if cols % cand == 0:
            acc_cols = cand
            break

    partials = _nmse_partials(p, t, steps_per_core, tile_r, acc_cols)
    return partials[0, 0, 0, 0]

    s = partials[:, :, 0, 0].sum(axis=0)
    inv_n = jnp.float32(1.0 / n)
    mse = s[0] * inv_n
    mean_t = s[1] * inv_n
    var = s[2] * inv_n - mean_t * mean_t
    return mse / var


# speedup vs baseline: 1.0480x; 1.0480x over previous
import functools

import jax
import jax.numpy as jnp
from jax.experimental import pallas as pl
from jax.experimental.pallas import tpu as pltpu

_LANES = 128
_SUB = 8
_TARGET_TILE_ELEMS = 1024 * 1024


def _nmse_kernel(p_ref, t_ref, o_ref, acc_ref, *, steps, tile_r, acc_cols,
                 n_elems):
    i = pl.program_id(0)

    cols = p_ref.shape[1]
    ssd = st = st2 = None
    first = True
    for k in range(tile_r // _SUB):
        band = slice(k * _SUB, (k + 1) * _SUB)
        for c in range(cols // acc_cols):
            cs = slice(c * acc_cols, (c + 1) * acc_cols)
            ps = p_ref[band, cs].astype(jnp.float32)
            ts = t_ref[band, cs].astype(jnp.float32)
            ds = ps - ts
            if first:
                ssd, st, st2 = ds * ds, ts, ts * ts
                first = False
            else:
                ssd += ds * ds
                st += ts
                st2 += ts * ts

    @pl.when(i == 0)
    def _start():
        acc_ref[0] = ssd
        acc_ref[1] = st
        acc_ref[2] = st2

    @pl.when(i > 0)
    def _accum():
        acc_ref[0] += ssd
        acc_ref[1] += st
        acc_ref[2] += st2

    @pl.when(i == steps - 1)
    def _emit():
        inv_n = jnp.float32(1.0 / n_elems)
        mse = jnp.sum(acc_ref[0]) * inv_n
        mean_t = jnp.sum(acc_ref[1]) * inv_n
        var = jnp.sum(acc_ref[2]) * inv_n - mean_t * mean_t
        o_ref[0, 0] = mse / var


def _nmse_scalar(p2d, t2d, steps, tile_r, acc_cols, n_elems):
    cols = p2d.shape[1]
    kernel_fn = functools.partial(
        _nmse_kernel, steps=steps, tile_r=tile_r, acc_cols=acc_cols,
        n_elems=n_elems)

    bytes_in = p2d.size * (p2d.dtype.itemsize + t2d.dtype.itemsize)
    return pl.pallas_call(
        kernel_fn,
        out_shape=jax.ShapeDtypeStruct((1, 1), jnp.float32),
        grid=(steps,),
        in_specs=[
            pl.BlockSpec((tile_r, cols), lambda i: (i, 0)),
            pl.BlockSpec((tile_r, cols), lambda i: (i, 0)),
        ],
        out_specs=pl.BlockSpec(memory_space=pltpu.SMEM),
        scratch_shapes=[pltpu.VMEM((3, _SUB, acc_cols), jnp.float32)],
        compiler_params=pltpu.CompilerParams(
            dimension_semantics=("arbitrary",),
        ),
        cost_estimate=pl.CostEstimate(
            flops=6 * n_elems, transcendentals=0, bytes_accessed=bytes_in),
    )(p2d, t2d)


def kernel(prediction: jax.Array, target: jax.Array) -> jax.Array:
    assert prediction.shape == target.shape
    n = prediction.size
    assert n > 0

    p = prediction
    t = target

    if not (p.ndim == 2 and p.shape[0] % _SUB == 0
            and p.shape[1] % _LANES == 0):
        p = p.reshape(-1)
        t = t.reshape(-1)
        chunk = _SUB * _LANES
        rem = p.size % chunk
        if rem:
            p = jnp.pad(p, (0, chunk - rem))
            t = jnp.pad(t, (0, chunk - rem))
        p = p.reshape(-1, _LANES)
        t = t.reshape(-1, _LANES)

    rows, cols = p.shape

    tile_r = _SUB
    tr = _SUB
    while tr <= rows:
        if rows % tr == 0:
            tile_r = tr
        tr *= 2
    while tile_r > _SUB and tile_r * cols > _TARGET_TILE_ELEMS:
        tile_r //= 2
    steps = rows // tile_r

    acc_cols = cols
    for cand in (1024, 512, 256, 128):
        if cols % cand == 0:
            acc_cols = cand
            break

    out = _nmse_scalar(p, t, steps, tile_r, acc_cols, n)
    return out.reshape(())
